# baseline (speedup 1.0000x reference)
import functools

import numpy as np
import jax
import jax.numpy as jnp
from jax import lax
from jax.experimental import pallas as pl
from jax.experimental.pallas import tpu as pltpu
from jax.experimental.pallas import tpu_sc as plsc

_NC = 2
_NS = 16
_NW = _NC * _NS
_LANES = 16


def _pos_encoding_np(length: int, d_model: int) -> np.ndarray:
    pos = np.arange(length)[:, np.newaxis].astype(np.float32)
    i = np.arange(d_model)[np.newaxis, :].astype(np.float32)
    angle_rates = 1.0 / np.power(
        10000.0, 2.0 * np.floor(i / 2.0) / np.float32(d_model))
    a = pos * angle_rates
    a[:, 0::2] = np.sin(a[:, 0::2])
    a[:, 1::2] = np.cos(a[:, 1::2])
    return a.astype(np.float32)


@functools.cache
def _make_sc_kernel(B: int, D: int):
    assert B % _NW == 0 and D % _LANES == 0
    b_per_w = B // _NW
    mesh = plsc.VectorSubcoreMesh(
        core_axis_name="c", subcore_axis_name="s",
        num_cores=_NC, num_subcores=_NS)

    @functools.partial(
        pl.kernel,
        out_type=jax.ShapeDtypeStruct((B, D), jnp.float32),
        mesh=mesh,
        scratch_types=[
            pltpu.VMEM((b_per_w,), jnp.int32),
            pltpu.VMEM((b_per_w, D), jnp.float32),
            pltpu.VMEM((b_per_w, D), jnp.float32),
            pltpu.SemaphoreType.DMA,
            pltpu.SemaphoreType.DMA,
        ],
        compiler_params=pltpu.CompilerParams(use_tc_tiling_on_sc=False),
    )
    def emb_kernel(table_hbm, idx_hbm, pos_hbm, out_hbm,
                   idx_v, rows_v, pos_v, gsem, psem):
        wid = lax.axis_index("s") * _NC + lax.axis_index("c")
        base = wid * b_per_w
        pltpu.sync_copy(idx_hbm.at[pl.ds(base, b_per_w)], idx_v)
        gather = pltpu.async_copy(table_hbm.at[idx_v], rows_v, gsem)
        pos_cp = pltpu.async_copy(
            pos_hbm.at[pl.ds(base, b_per_w)], pos_v, psem)
        gather.wait()
        pos_cp.wait()

        def add_row(r, carry):
            for j in range(D // _LANES):
                sl = pl.ds(j * _LANES, _LANES)
                rows_v[r, sl] = rows_v[r, sl] + pos_v[r, sl]
            return carry

        lax.fori_loop(0, b_per_w, add_row, 0, unroll=4)
        pltpu.sync_copy(rows_v, out_hbm.at[pl.ds(base, b_per_w)])

    return emb_kernel


def kernel(x, table):
    length = x.shape[0]
    d_model = table.shape[1]
    pos = jnp.asarray(_pos_encoding_np(length, d_model))
    out = _make_sc_kernel(length, d_model)(table, x.astype(jnp.int32), pos)
    return jnp.reshape(out, (1, length, d_model))


# speedup vs baseline: 1.7184x; 1.7184x over previous
import functools

import numpy as np
import jax
import jax.numpy as jnp
from jax import lax
from jax.experimental import pallas as pl
from jax.experimental.pallas import tpu as pltpu
from jax.experimental.pallas import tpu_sc as plsc

_NC = 2
_NS = 16
_NW = _NC * _NS
_LANES = 16


def _pos_encoding_np(length: int, d_model: int) -> np.ndarray:
    pos = np.arange(length)[:, np.newaxis].astype(np.float32)
    i = np.arange(d_model)[np.newaxis, :].astype(np.float32)
    angle_rates = 1.0 / np.power(
        10000.0, 2.0 * np.floor(i / 2.0) / np.float32(d_model))
    a = pos * angle_rates
    a[:, 0::2] = np.sin(a[:, 0::2])
    a[:, 1::2] = np.cos(a[:, 1::2])
    return a.astype(np.float32)


@functools.cache
def _make_sc_kernel(B: int, D: int):
    assert B % _NW == 0 and D % _LANES == 0
    b_per_w = B // _NW
    mesh = plsc.VectorSubcoreMesh(
        core_axis_name="c", subcore_axis_name="s",
        num_cores=_NC, num_subcores=_NS)

    @functools.partial(
        pl.kernel,
        out_type=jax.ShapeDtypeStruct((B, D), jnp.float32),
        mesh=mesh,
        scratch_types=[
            pltpu.VMEM((b_per_w,), jnp.int32),
            pltpu.VMEM((b_per_w, D), jnp.float32),
            pltpu.VMEM((b_per_w, D), jnp.float32),
            pltpu.SemaphoreType.DMA,
            pltpu.SemaphoreType.DMA,
        ],
    )
    def emb_kernel(table_hbm, idx_hbm, pos_hbm, out_hbm,
                   idx_v, rows_v, pos_v, gsem, psem):
        wid = lax.axis_index("s") * _NC + lax.axis_index("c")
        base = wid * b_per_w
        pltpu.sync_copy(idx_hbm.at[pl.ds(base, b_per_w)], idx_v)
        pos_cp = pltpu.async_copy(
            pos_hbm.at[pl.ds(base, b_per_w)], pos_v, psem)

        def fire_group(g, carry):
            ids = idx_v[pl.ds(g * _LANES, _LANES)]
            for l in range(_LANES):
                pltpu.make_async_copy(
                    table_hbm.at[pl.ds(ids[l], 1)],
                    rows_v.at[pl.ds(g * _LANES + l, 1)],
                    gsem).start()
            return carry

        lax.fori_loop(0, b_per_w // _LANES, fire_group, 0)
        pltpu.make_async_copy(
            table_hbm.at[pl.ds(0, b_per_w)], rows_v, gsem).wait()
        pos_cp.wait()

        def add_row(r, carry):
            for j in range(D // _LANES):
                sl = pl.ds(j * _LANES, _LANES)
                rows_v[r, sl] = rows_v[r, sl] + pos_v[r, sl]
            return carry

        lax.fori_loop(0, b_per_w, add_row, 0, unroll=4)
        pltpu.sync_copy(rows_v, out_hbm.at[pl.ds(base, b_per_w)])

    return emb_kernel


def kernel(x, table):
    length = x.shape[0]
    d_model = table.shape[1]
    pos = jnp.asarray(_pos_encoding_np(length, d_model))
    out = _make_sc_kernel(length, d_model)(table, x.astype(jnp.int32), pos)
    return jnp.reshape(out, (1, length, d_model))


# speedup vs baseline: 1.7227x; 1.0025x over previous
import functools

import numpy as np
import jax
import jax.numpy as jnp
from jax import lax
from jax.experimental import pallas as pl
from jax.experimental.pallas import tpu as pltpu
from jax.experimental.pallas import tpu_sc as plsc

_NC = 2
_NS = 16
_NW = _NC * _NS
_LANES = 16


def _pos_encoding_np(length: int, d_model: int) -> np.ndarray:
    pos = np.arange(length)[:, np.newaxis].astype(np.float32)
    i = np.arange(d_model)[np.newaxis, :].astype(np.float32)
    angle_rates = 1.0 / np.power(
        10000.0, 2.0 * np.floor(i / 2.0) / np.float32(d_model))
    a = pos * angle_rates
    a[:, 0::2] = np.sin(a[:, 0::2])
    a[:, 1::2] = np.cos(a[:, 1::2])
    return a.astype(np.float32)


@functools.cache
def _make_sc_kernel(B: int, D: int):
    assert B % _NW == 0 and D % _LANES == 0
    b_per_w = B // _NW
    mesh = plsc.VectorSubcoreMesh(
        core_axis_name="c", subcore_axis_name="s",
        num_cores=_NC, num_subcores=_NS)

    @functools.partial(
        pl.kernel,
        out_type=jax.ShapeDtypeStruct((B, D), jnp.float32),
        mesh=mesh,
        scratch_types=[
            pltpu.VMEM((b_per_w,), jnp.int32),
            pltpu.VMEM((b_per_w, D), jnp.float32),
            pltpu.VMEM((b_per_w, D), jnp.float32),
            pltpu.SemaphoreType.DMA,
            pltpu.SemaphoreType.DMA,
        ],
        compiler_params=pltpu.CompilerParams(use_tc_tiling_on_sc=True),
    )
    def emb_kernel(table_hbm, idx_hbm, pos_hbm, out_hbm,
                   idx_v, rows_v, pos_v, gsem, psem):
        wid = lax.axis_index("s") * _NC + lax.axis_index("c")
        base = wid * b_per_w
        pltpu.sync_copy(idx_hbm.at[pl.ds(base, b_per_w)], idx_v)
        pos_cp = pltpu.async_copy(
            pos_hbm.at[pl.ds(base, b_per_w)], pos_v, psem)

        def fire_group(g, carry):
            ids = idx_v[pl.ds(g * _LANES, _LANES)]
            for l in range(_LANES):
                pltpu.make_async_copy(
                    table_hbm.at[pl.ds(ids[l], 1)],
                    rows_v.at[pl.ds(g * _LANES + l, 1)],
                    gsem).start()
            return carry

        lax.fori_loop(0, b_per_w // _LANES, fire_group, 0)
        pltpu.make_async_copy(
            table_hbm.at[pl.ds(0, b_per_w)], rows_v, gsem).wait()
        pos_cp.wait()

        def add_row(r, carry):
            for j in range(D // _LANES):
                sl = pl.ds(j * _LANES, _LANES)
                rows_v[r, sl] = rows_v[r, sl] + pos_v[r, sl]
            return carry

        lax.fori_loop(0, b_per_w, add_row, 0, unroll=4)
        pltpu.sync_copy(rows_v, out_hbm.at[pl.ds(base, b_per_w)])

    return emb_kernel


def kernel(x, table):
    length = x.shape[0]
    d_model = table.shape[1]
    pos = jnp.asarray(_pos_encoding_np(length, d_model))
    out = _make_sc_kernel(length, d_model)(table, x.astype(jnp.int32), pos)
    return jnp.reshape(out, (1, length, d_model))


# speedup vs baseline: 9.0924x; 5.2779x over previous
import functools

import numpy as np
import jax
import jax.numpy as jnp
from jax import lax
from jax.experimental import pallas as pl
from jax.experimental.pallas import tpu as pltpu
from jax.experimental.pallas import tpu_sc as plsc

_NC = 2
_NS = 16
_NW = _NC * _NS
_LANES = 16
_BLK = 128
_NBUF = 8


def _pos_encoding_np(length: int, d_model: int) -> np.ndarray:
    pos = np.arange(length)[:, np.newaxis].astype(np.float32)
    i = np.arange(d_model)[np.newaxis, :].astype(np.float32)
    angle_rates = 1.0 / np.power(
        10000.0, 2.0 * np.floor(i / 2.0) / np.float32(d_model))
    a = pos * angle_rates
    a[:, 0::2] = np.sin(a[:, 0::2])
    a[:, 1::2] = np.cos(a[:, 1::2])
    return a.astype(np.float32)


@functools.cache
def _make_sc_kernel(B: int, D: int):
    assert B % _NW == 0 and D % _LANES == 0
    b_per_w = B // _NW
    assert b_per_w % _LANES == 0
    n_groups = b_per_w // _LANES
    mesh = plsc.VectorSubcoreMesh(
        core_axis_name="c", subcore_axis_name="s",
        num_cores=_NC, num_subcores=_NS)
    @functools.partial(
        pl.kernel,
        out_type=jax.ShapeDtypeStruct((D, B), jnp.float32),
        mesh=mesh,
        scratch_types=[
            pltpu.VMEM((b_per_w,), jnp.int32),
            pltpu.VMEM((D, b_per_w), jnp.float32),
            pltpu.VMEM((_NBUF, D, _BLK), jnp.float32),
            pltpu.SemaphoreType.DMA((_NBUF,)),
            pltpu.SemaphoreType.DMA,
        ],
        compiler_params=pltpu.CompilerParams(needs_layout_passes=False),
    )
    def emb_kernel(tab_t, idx_hbm, pos_t, out_t,
                   idx_v, acc_v, bufs, gsem, psem):
        wid = lax.axis_index("s") * _NC + lax.axis_index("c")
        base = wid * b_per_w
        pltpu.sync_copy(idx_hbm.at[pl.ds(base, b_per_w)], idx_v)
        pos_cp = pltpu.async_copy(
            pos_t.at[:, pl.ds(base, b_per_w)], acc_v, psem)

        def fire(ids16, l, slot):
            blk = ids16[l] >> 7
            off = pl.multiple_of(blk * _BLK, _BLK)
            pltpu.make_async_copy(
                tab_t.at[:, pl.ds(off, _BLK)],
                bufs.at[slot], gsem.at[slot]).start()

        def extract(m16, l, slot, col):
            pltpu.make_async_copy(
                tab_t.at[:, pl.ds(0, _BLK)],
                bufs.at[slot], gsem.at[slot]).wait()
            m = jnp.full((_LANES,), m16[l], dtype=jnp.int32)
            s = jnp.full((_LANES,), slot, dtype=jnp.int32)
            c = jnp.full((_LANES,), col, dtype=jnp.int32)
            lanes = lax.iota(jnp.int32, _LANES)
            for jc in range(D // _LANES):
                rows = lanes + jc * _LANES
                vals = plsc.load_gather(bufs, [s, rows, m])
                plsc.addupdate_scatter(acc_v, [rows, c], vals)

        ids0 = idx_v[pl.ds(0, _LANES)]
        for l in range(_NBUF):
            fire(ids0, l, l)

        pos_cp.wait()

        def group(g, carry):
            ids16 = idx_v[pl.ds(g * _LANES, _LANES)]
            m16 = ids16 & (_BLK - 1)
            ids_n = idx_v[pl.ds((g + 1) * _LANES, _LANES)]
            for l in range(_LANES):
                slot = l % _NBUF
                extract(m16, l, slot, g * _LANES + l)
                if l < _LANES - _NBUF:
                    fire(ids16, l + _NBUF, slot)
                else:
                    fire(ids_n, l - (_LANES - _NBUF), slot)
            return carry

        lax.fori_loop(0, n_groups - 1, group, 0)

        gl = n_groups - 1
        ids_l = idx_v[pl.ds(gl * _LANES, _LANES)]
        m_l = ids_l & (_BLK - 1)
        for l in range(_LANES):
            slot = l % _NBUF
            extract(m_l, l, slot, gl * _LANES + l)
            if l < _LANES - _NBUF:
                fire(ids_l, l + _NBUF, slot)

        pltpu.sync_copy(acc_v, out_t.at[:, pl.ds(base, b_per_w)])

    return emb_kernel


def kernel(x, table):
    length = x.shape[0]
    d_model = table.shape[1]
    pos_t = jnp.asarray(
        np.ascontiguousarray(_pos_encoding_np(length, d_model).T))
    out_t = _make_sc_kernel(length, d_model)(
        table.T, x.astype(jnp.int32), pos_t)
    return jnp.reshape(out_t.T, (1, length, d_model))
